# baseline (speedup 1.0000x reference)
import functools

import jax
import jax.numpy as jnp
from jax import lax
from jax.experimental import pallas as pl
from jax.experimental.pallas import tpu as pltpu
from jax.experimental.pallas import tpu_sc as plsc

B = 4096
D = 512
NUM_BINS = 32
TAIL_BOUND = 3.0
MIN_BIN_WIDTH = 1e-3
MIN_BIN_HEIGHT = 1e-3
MIN_DERIVATIVE = 1e-3

NC = 2
NS = 16
L = 16
NW = NC * NS
ROWS_PER_TILE = B // NW
R = 16
NCHUNK = ROWS_PER_TILE // R
NVEC = D // L

LN2 = 0.6931471805599453


def _cumsum_lanes(x):
    n = x.shape[-1]
    shift = 1
    while shift < n:
        pad = jnp.zeros(x.shape[:-1] + (shift,), x.dtype)
        x = x + jnp.concatenate([pad, x[..., :-shift]], axis=-1)
        shift *= 2
    return x


def _normalized_cum(unnorm, min_frac):
    m = jnp.max(unnorm, axis=-1, keepdims=True)
    e = jnp.exp(unnorm - m)
    w = e / jnp.sum(e, axis=-1, keepdims=True)
    w = min_frac + (1.0 - min_frac * NUM_BINS) * w
    cs = _cumsum_lanes(w)
    full = jnp.concatenate([jnp.zeros((D, 1), jnp.float32), cs], axis=-1)
    full = 2.0 * TAIL_BOUND * full - TAIL_BOUND
    col = jax.lax.broadcasted_iota(jnp.int32, full.shape, 1)
    full = jnp.where(col == 0, -TAIL_BOUND, full)
    full = jnp.where(col == NUM_BINS, TAIL_BOUND, full)
    return full


def _prep_body(uw_ref, uh_ref, ud_ref,
               edges_ref, w_ref, ch_ref, dl_ref, d0_ref, d1_ref):
    cw_full = _normalized_cum(uw_ref[...], MIN_BIN_WIDTH)
    ch_full = _normalized_cum(uh_ref[...], MIN_BIN_HEIGHT)
    widths = cw_full[:, 1:] - cw_full[:, :-1]
    heights = ch_full[:, 1:] - ch_full[:, :-1]
    delta = heights / widths
    sp = MIN_DERIVATIVE + jnp.log1p(jnp.exp(ud_ref[...]))
    ones = jnp.ones((D, 1), jnp.float32)
    d_full = jnp.concatenate([ones, sp, ones], axis=-1)
    edges_ref[...] = cw_full[:, :NUM_BINS].T
    w_ref[...] = widths.T
    ch_ref[...] = ch_full[:, :NUM_BINS].T
    dl_ref[...] = delta.T
    d0_ref[...] = d_full[:, :NUM_BINS].T
    d1_ref[...] = d_full[:, 1:].T


def _prep_tables(uw, uh, ud):
    outs = [jax.ShapeDtypeStruct((NUM_BINS, D), jnp.float32)] * 6
    return pl.pallas_call(
        _prep_body,
        out_shape=outs,
    )(uw, uh, ud)


def _ln_1_2(m):
    adj = m > 1.4142135623730951
    mm = jnp.where(adj, 0.5 * m, m)
    ee = jnp.where(adj, 1.0, 0.0)
    z = mm - 1.0
    p = jnp.float32(7.0376836292e-2)
    p = p * z - 1.1514610310e-1
    p = p * z + 1.1676998740e-1
    p = p * z - 1.2420140846e-1
    p = p * z + 1.4249322787e-1
    p = p * z - 1.6668057665e-1
    p = p * z + 2.0000714765e-1
    p = p * z - 2.4999993993e-1
    p = p * z + 3.3333331174e-1
    z2 = z * z
    y = z2 * z * p - 0.5 * z2
    return z + y + ee * LN2


def _sc_main(x, tables):
    edges_t, w_t, ch_t, dl_t, d0_t, d1_t = tables
    mesh = plsc.VectorSubcoreMesh(core_axis_name="c", subcore_axis_name="s")

    @functools.partial(
        pl.kernel,
        out_type=[jax.ShapeDtypeStruct((B, D), jnp.float32),
                  jax.ShapeDtypeStruct((B,), jnp.float32)],
        mesh=mesh,
        compiler_params=pltpu.CompilerParams(use_tc_tiling_on_sc=False,
                                             needs_layout_passes=False),
        scratch_types=[
            pltpu.VMEM((NUM_BINS, D), jnp.float32),
            pltpu.VMEM((NUM_BINS, D), jnp.float32),
            pltpu.VMEM((NUM_BINS, D), jnp.float32),
            pltpu.VMEM((NUM_BINS, D), jnp.float32),
            pltpu.VMEM((NUM_BINS, D), jnp.float32),
            pltpu.VMEM((NUM_BINS, D), jnp.float32),
            pltpu.VMEM((R, D), jnp.float32),
            pltpu.VMEM((R, D), jnp.float32),
            pltpu.VMEM((R,), jnp.float32),
        ],
    )
    def sc_kernel(x_hbm, edges_hbm, w_hbm, ch_hbm, dl_hbm, d0_hbm, d1_hbm,
                  out_hbm, lad_hbm,
                  edges_v, w_v, ch_v, dl_v, d0_v, d1_v, xbuf, obuf, ladbuf):
        wid = lax.axis_index("s") * NC + lax.axis_index("c")
        row0 = wid * ROWS_PER_TILE
        pltpu.sync_copy(edges_hbm, edges_v)
        pltpu.sync_copy(w_hbm, w_v)
        pltpu.sync_copy(ch_hbm, ch_v)
        pltpu.sync_copy(dl_hbm, dl_v)
        pltpu.sync_copy(d0_hbm, d0_v)
        pltpu.sync_copy(d1_hbm, d1_v)

        lane = lax.iota(jnp.int32, L)

        def vec_body(r, j, carry):
            pacc, eacc = carry
            x0 = xbuf[r, pl.ds(j * L, L)]
            inside = (x0 >= -TAIL_BOUND) & (x0 <= TAIL_BOUND)
            xc = jnp.minimum(jnp.maximum(x0, -TAIL_BOUND), TAIL_BOUND)
            dvec = lane + j * L
            lo = jnp.zeros((L,), jnp.int32)
            for s in (16, 8, 4, 2, 1):
                probe = lo + s
                ev = plsc.load_gather(edges_v, [probe, dvec])
                lo = jnp.where(xc >= ev, probe, lo)
            cw = plsc.load_gather(edges_v, [lo, dvec])
            w = plsc.load_gather(w_v, [lo, dvec])
            chh = plsc.load_gather(ch_v, [lo, dvec])
            dl = plsc.load_gather(dl_v, [lo, dvec])
            da = plsc.load_gather(d0_v, [lo, dvec])
            db = plsc.load_gather(d1_v, [lo, dvec])

            theta = (xc - cw) / w
            t2 = theta * theta
            t1m = theta - t2
            num = (dl * w) * (dl * t2 + da * t1m)
            den = dl + (da + db - 2.0 * dl) * t1m
            inv = 1.0 / den
            out_s = chh + num * inv
            omt = 1.0 - theta
            dnum = (dl * dl) * (db * t2 + 2.0 * dl * t1m + da * omt * omt)
            obuf[r, pl.ds(j * L, L)] = jnp.where(inside, out_s, x0)

            ratio = jnp.where(inside, dnum * inv * inv, 1.0)
            pacc = pacc * ratio
            bits = plsc.bitcast(pacc, jnp.int32)
            eacc = eacc + (lax.shift_right_arithmetic(bits, 23) - 127)
            mant = (bits & 0x007FFFFF) | 0x3F800000
            pacc = plsc.bitcast(mant, jnp.float32)
            return pacc, eacc

        def row_body(r, lad_acc):
            init = (jnp.ones((L,), jnp.float32), jnp.zeros((L,), jnp.int32))

            @plsc.parallel_loop(0, NVEC, carry=init, unroll=8)
            def final_carry(j, c):
                return vec_body(r, j, c)

            pacc, eacc = final_carry
            lad = _ln_1_2(pacc) + eacc.astype(jnp.float32) * LN2
            s = jnp.sum(lad)
            return lad_acc + jnp.where(lane == r, s, 0.0)

        def chunk_body(c, _):
            base = row0 + c * R
            pltpu.sync_copy(x_hbm.at[pl.ds(base, R)], xbuf)
            lad_acc = lax.fori_loop(0, R, row_body,
                                    jnp.zeros((L,), jnp.float32))
            ladbuf[...] = lad_acc
            pltpu.sync_copy(obuf, out_hbm.at[pl.ds(base, R)])
            pltpu.sync_copy(ladbuf, lad_hbm.at[pl.ds(base, R)])
            return 0

        lax.fori_loop(0, NCHUNK, chunk_body, 0)

    return sc_kernel(x, edges_t, w_t, ch_t, dl_t, d0_t, d1_t)


@jax.jit
def kernel(inputs, unnormalized_widths, unnormalized_heights,
           unnormalized_derivatives):
    tables = _prep_tables(unnormalized_widths, unnormalized_heights,
                          unnormalized_derivatives)
    out, lad = _sc_main(inputs, tables)
    return out, lad


# speedup vs baseline: 1.2341x; 1.2341x over previous
import functools

import jax
import jax.numpy as jnp
from jax import lax
from jax.experimental import pallas as pl
from jax.experimental.pallas import tpu as pltpu
from jax.experimental.pallas import tpu_sc as plsc

B = 4096
D = 512
NUM_BINS = 32
TAIL_BOUND = 3.0
MIN_BIN_WIDTH = 1e-3
MIN_BIN_HEIGHT = 1e-3
MIN_DERIVATIVE = 1e-3

NC = 2
NS = 16
L = 16
NW = NC * NS
ROWS_PER_TILE = B // NW
R = 16
NCHUNK = ROWS_PER_TILE // R
NVEC = D // L

LN2 = 0.6931471805599453


def _cumsum_lanes(x):
    n = x.shape[-1]
    shift = 1
    while shift < n:
        pad = jnp.zeros(x.shape[:-1] + (shift,), x.dtype)
        x = x + jnp.concatenate([pad, x[..., :-shift]], axis=-1)
        shift *= 2
    return x


def _normalized_cum(unnorm, min_frac):
    m = jnp.max(unnorm, axis=-1, keepdims=True)
    e = jnp.exp(unnorm - m)
    w = e / jnp.sum(e, axis=-1, keepdims=True)
    w = min_frac + (1.0 - min_frac * NUM_BINS) * w
    cs = _cumsum_lanes(w)
    full = jnp.concatenate([jnp.zeros((D, 1), jnp.float32), cs], axis=-1)
    full = 2.0 * TAIL_BOUND * full - TAIL_BOUND
    col = jax.lax.broadcasted_iota(jnp.int32, full.shape, 1)
    full = jnp.where(col == 0, -TAIL_BOUND, full)
    full = jnp.where(col == NUM_BINS, TAIL_BOUND, full)
    return full


def _prep_body(uw_ref, uh_ref, ud_ref,
               edges_ref, w_ref, ch_ref, dl_ref, d0_ref, d1_ref):
    cw_full = _normalized_cum(uw_ref[...], MIN_BIN_WIDTH)
    ch_full = _normalized_cum(uh_ref[...], MIN_BIN_HEIGHT)
    widths = cw_full[:, 1:] - cw_full[:, :-1]
    heights = ch_full[:, 1:] - ch_full[:, :-1]
    delta = heights / widths
    sp = MIN_DERIVATIVE + jnp.log1p(jnp.exp(ud_ref[...]))
    ones = jnp.ones((D, 1), jnp.float32)
    d_full = jnp.concatenate([ones, sp, ones], axis=-1)
    edges_ref[...] = cw_full[:, :NUM_BINS].T
    w_ref[...] = widths.T
    ch_ref[...] = ch_full[:, :NUM_BINS].T
    dl_ref[...] = delta.T
    d0_ref[...] = d_full[:, :NUM_BINS].T
    d1_ref[...] = d_full[:, 1:].T


def _prep_tables(uw, uh, ud):
    outs = [jax.ShapeDtypeStruct((NUM_BINS, D), jnp.float32)] * 6
    return pl.pallas_call(
        _prep_body,
        out_shape=outs,
    )(uw, uh, ud)


def _ln_1_2(m):
    adj = m > 1.4142135623730951
    mm = jnp.where(adj, 0.5 * m, m)
    ee = jnp.where(adj, 1.0, 0.0)
    z = mm - 1.0
    p = jnp.float32(7.0376836292e-2)
    p = p * z - 1.1514610310e-1
    p = p * z + 1.1676998740e-1
    p = p * z - 1.2420140846e-1
    p = p * z + 1.4249322787e-1
    p = p * z - 1.6668057665e-1
    p = p * z + 2.0000714765e-1
    p = p * z - 2.4999993993e-1
    p = p * z + 3.3333331174e-1
    z2 = z * z
    y = z2 * z * p - 0.5 * z2
    return z + y + ee * LN2


def _sc_main(x, tables):
    edges_t, w_t, ch_t, dl_t, d0_t, d1_t = tables
    mesh = plsc.VectorSubcoreMesh(core_axis_name="c", subcore_axis_name="s")

    @functools.partial(
        pl.kernel,
        out_type=[jax.ShapeDtypeStruct((B, D), jnp.float32),
                  jax.ShapeDtypeStruct((B,), jnp.float32)],
        mesh=mesh,
        compiler_params=pltpu.CompilerParams(use_tc_tiling_on_sc=False,
                                             needs_layout_passes=False),
        scratch_types=[
            pltpu.VMEM((NUM_BINS, D), jnp.float32),
            pltpu.VMEM((NUM_BINS, D), jnp.float32),
            pltpu.VMEM((NUM_BINS, D), jnp.float32),
            pltpu.VMEM((NUM_BINS, D), jnp.float32),
            pltpu.VMEM((NUM_BINS, D), jnp.float32),
            pltpu.VMEM((NUM_BINS, D), jnp.float32),
            pltpu.VMEM((R, D), jnp.float32),
            pltpu.VMEM((R, D), jnp.float32),
            pltpu.VMEM((R,), jnp.float32),
        ],
    )
    def sc_kernel(x_hbm, edges_hbm, w_hbm, ch_hbm, dl_hbm, d0_hbm, d1_hbm,
                  out_hbm, lad_hbm,
                  edges_v, w_v, ch_v, dl_v, d0_v, d1_v, xbuf, obuf, ladbuf):
        wid = lax.axis_index("s") * NC + lax.axis_index("c")
        row0 = wid * ROWS_PER_TILE
        pltpu.sync_copy(edges_hbm, edges_v)
        pltpu.sync_copy(w_hbm, w_v)
        pltpu.sync_copy(ch_hbm, ch_v)
        pltpu.sync_copy(dl_hbm, dl_v)
        pltpu.sync_copy(d0_hbm, d0_v)
        pltpu.sync_copy(d1_hbm, d1_v)

        lane = lax.iota(jnp.int32, L)

        def vec_body(r, j, carry):
            pacc, eacc = carry
            x0 = xbuf[r, pl.ds(j * L, L)]
            inside = (x0 >= -TAIL_BOUND) & (x0 <= TAIL_BOUND)
            xc = jnp.minimum(jnp.maximum(x0, -TAIL_BOUND), TAIL_BOUND)
            dvec = lane + j * L
            lo = jnp.zeros((L,), jnp.int32)
            for s in (16, 8, 4, 2, 1):
                probe = lo + s
                ev = plsc.load_gather(edges_v, [probe, dvec])
                lo = jnp.where(xc >= ev, probe, lo)
            cw = plsc.load_gather(edges_v, [lo, dvec])
            w = plsc.load_gather(w_v, [lo, dvec])
            chh = plsc.load_gather(ch_v, [lo, dvec])
            dl = plsc.load_gather(dl_v, [lo, dvec])
            da = plsc.load_gather(d0_v, [lo, dvec])
            db = plsc.load_gather(d1_v, [lo, dvec])

            theta = (xc - cw) / w
            t2 = theta * theta
            t1m = theta - t2
            num = (dl * w) * (dl * t2 + da * t1m)
            den = dl + (da + db - 2.0 * dl) * t1m
            inv = 1.0 / den
            out_s = chh + num * inv
            omt = 1.0 - theta
            dnum = (dl * dl) * (db * t2 + 2.0 * dl * t1m + da * omt * omt)
            obuf[r, pl.ds(j * L, L)] = jnp.where(inside, out_s, x0)

            ratio = jnp.where(inside, dnum * inv * inv, 1.0)
            pacc = pacc * ratio
            bits = plsc.bitcast(pacc, jnp.int32)
            eacc = eacc + (lax.shift_right_arithmetic(bits, 23) - 127)
            mant = (bits & 0x007FFFFF) | 0x3F800000
            pacc = plsc.bitcast(mant, jnp.float32)
            return pacc, eacc

        def row_body(r, lad_acc):
            init = (jnp.ones((L,), jnp.float32), jnp.zeros((L,), jnp.int32))

            @plsc.parallel_loop(0, NVEC, carry=init, unroll=4)
            def final_carry(j, c):
                return vec_body(r, j, c)

            pacc, eacc = final_carry
            lad = _ln_1_2(pacc) + eacc.astype(jnp.float32) * LN2
            s = jnp.sum(lad)
            return lad_acc + jnp.where(lane == r, s, 0.0)

        def chunk_body(c, _):
            base = row0 + c * R
            pltpu.sync_copy(x_hbm.at[pl.ds(base, R)], xbuf)
            lad_acc = lax.fori_loop(0, R, row_body,
                                    jnp.zeros((L,), jnp.float32))
            ladbuf[...] = lad_acc
            pltpu.sync_copy(obuf, out_hbm.at[pl.ds(base, R)])
            pltpu.sync_copy(ladbuf, lad_hbm.at[pl.ds(base, R)])
            return 0

        lax.fori_loop(0, NCHUNK, chunk_body, 0)

    return sc_kernel(x, edges_t, w_t, ch_t, dl_t, d0_t, d1_t)


@jax.jit
def kernel(inputs, unnormalized_widths, unnormalized_heights,
           unnormalized_derivatives):
    tables = _prep_tables(unnormalized_widths, unnormalized_heights,
                          unnormalized_derivatives)
    out, lad = _sc_main(inputs, tables)
    return out, lad


# speedup vs baseline: 1.3063x; 1.0586x over previous
import functools

import jax
import jax.numpy as jnp
from jax import lax
from jax.experimental import pallas as pl
from jax.experimental.pallas import tpu as pltpu
from jax.experimental.pallas import tpu_sc as plsc

B = 4096
D = 512
NUM_BINS = 32
TAIL_BOUND = 3.0
MIN_BIN_WIDTH = 1e-3
MIN_BIN_HEIGHT = 1e-3
MIN_DERIVATIVE = 1e-3

NC = 2
NS = 16
L = 16
NW = NC * NS
ROWS_PER_TILE = B // NW
R = 16
NCHUNK = ROWS_PER_TILE // R
NVEC = D // L

LN2 = 0.6931471805599453


def _cumsum_lanes(x):
    n = x.shape[-1]
    shift = 1
    while shift < n:
        pad = jnp.zeros(x.shape[:-1] + (shift,), x.dtype)
        x = x + jnp.concatenate([pad, x[..., :-shift]], axis=-1)
        shift *= 2
    return x


def _normalized_cum(unnorm, min_frac):
    m = jnp.max(unnorm, axis=-1, keepdims=True)
    e = jnp.exp(unnorm - m)
    w = e / jnp.sum(e, axis=-1, keepdims=True)
    w = min_frac + (1.0 - min_frac * NUM_BINS) * w
    cs = _cumsum_lanes(w)
    full = jnp.concatenate([jnp.zeros((D, 1), jnp.float32), cs], axis=-1)
    full = 2.0 * TAIL_BOUND * full - TAIL_BOUND
    col = jax.lax.broadcasted_iota(jnp.int32, full.shape, 1)
    full = jnp.where(col == 0, -TAIL_BOUND, full)
    full = jnp.where(col == NUM_BINS, TAIL_BOUND, full)
    return full


def _prep_body(uw_ref, uh_ref, ud_ref,
               edges_ref, ch_ref, dl_ref, dv_ref):
    cw_full = _normalized_cum(uw_ref[...], MIN_BIN_WIDTH)
    ch_full = _normalized_cum(uh_ref[...], MIN_BIN_HEIGHT)
    widths = cw_full[:, 1:] - cw_full[:, :-1]
    heights = ch_full[:, 1:] - ch_full[:, :-1]
    delta = heights / widths
    sp = MIN_DERIVATIVE + jnp.log1p(jnp.exp(ud_ref[...]))
    ones = jnp.ones((D, 1), jnp.float32)
    d_full = jnp.concatenate([ones, sp, ones], axis=-1)
    edges_ref[...] = cw_full.T
    ch_ref[...] = ch_full[:, :NUM_BINS].T
    dl_ref[...] = delta.T
    dv_ref[...] = d_full.T


def _prep_tables(uw, uh, ud):
    outs = [jax.ShapeDtypeStruct((NUM_BINS + 1, D), jnp.float32),
            jax.ShapeDtypeStruct((NUM_BINS, D), jnp.float32),
            jax.ShapeDtypeStruct((NUM_BINS, D), jnp.float32),
            jax.ShapeDtypeStruct((NUM_BINS + 1, D), jnp.float32)]
    return pl.pallas_call(
        _prep_body,
        out_shape=outs,
    )(uw, uh, ud)


def _ln_1_2(m):
    adj = m > 1.4142135623730951
    mm = jnp.where(adj, 0.5 * m, m)
    ee = jnp.where(adj, 1.0, 0.0)
    z = mm - 1.0
    p = jnp.float32(7.0376836292e-2)
    p = p * z - 1.1514610310e-1
    p = p * z + 1.1676998740e-1
    p = p * z - 1.2420140846e-1
    p = p * z + 1.4249322787e-1
    p = p * z - 1.6668057665e-1
    p = p * z + 2.0000714765e-1
    p = p * z - 2.4999993993e-1
    p = p * z + 3.3333331174e-1
    z2 = z * z
    y = z2 * z * p - 0.5 * z2
    return z + y + ee * LN2


def _sc_main(x, tables):
    edges_t, ch_t, dl_t, dv_t = tables
    mesh = plsc.VectorSubcoreMesh(core_axis_name="c", subcore_axis_name="s")

    @functools.partial(
        pl.kernel,
        out_type=[jax.ShapeDtypeStruct((B, D), jnp.float32),
                  jax.ShapeDtypeStruct((B,), jnp.float32)],
        mesh=mesh,
        compiler_params=pltpu.CompilerParams(use_tc_tiling_on_sc=False,
                                             needs_layout_passes=False),
        scratch_types=[
            pltpu.VMEM((NUM_BINS + 1, D), jnp.float32),
            pltpu.VMEM((NUM_BINS, D), jnp.float32),
            pltpu.VMEM((NUM_BINS, D), jnp.float32),
            pltpu.VMEM((NUM_BINS + 1, D), jnp.float32),
            pltpu.VMEM((2, R, D), jnp.float32),
            pltpu.VMEM((2, R, D), jnp.float32),
            pltpu.VMEM((2, R), jnp.float32),
            pltpu.SemaphoreType.DMA,
            pltpu.SemaphoreType.DMA,
            pltpu.SemaphoreType.DMA,
            pltpu.SemaphoreType.DMA,
            pltpu.SemaphoreType.DMA,
        ],
    )
    def sc_kernel(x_hbm, edges_hbm, ch_hbm, dl_hbm, dv_hbm,
                  out_hbm, lad_hbm,
                  edges_v, ch_v, dl_v, dv_v, xbuf, obuf, ladbuf,
                  in_sem0, in_sem1, out_sem0, out_sem1, tab_sem):
        wid = lax.axis_index("s") * NC + lax.axis_index("c")
        row0 = wid * ROWS_PER_TILE
        in_sems = (in_sem0, in_sem1)
        out_sems = (out_sem0, out_sem1)
        tab_copies = [
            pltpu.async_copy(edges_hbm, edges_v, tab_sem),
            pltpu.async_copy(ch_hbm, ch_v, tab_sem),
            pltpu.async_copy(dl_hbm, dl_v, tab_sem),
            pltpu.async_copy(dv_hbm, dv_v, tab_sem),
        ]
        first_in = pltpu.async_copy(
            x_hbm.at[pl.ds(row0, R)], xbuf.at[0], in_sems[0])
        for cp in tab_copies:
            cp.wait()

        lane = lax.iota(jnp.int32, L)

        def vec_body(xb, ob, r, j, carry):
            pacc, eacc = carry
            x0 = xb[r, pl.ds(j * L, L)]
            inside = (x0 >= -TAIL_BOUND) & (x0 <= TAIL_BOUND)
            xc = jnp.minimum(jnp.maximum(x0, -TAIL_BOUND), TAIL_BOUND)
            dvec = lane + j * L
            lo = jnp.zeros((L,), jnp.int32)
            for s in (16, 8, 4, 2, 1):
                probe = lo + s
                ev = plsc.load_gather(edges_v, [probe, dvec])
                lo = jnp.where(xc >= ev, probe, lo)
            lo1 = lo + 1
            cw = plsc.load_gather(edges_v, [lo, dvec])
            cw1 = plsc.load_gather(edges_v, [lo1, dvec])
            chh = plsc.load_gather(ch_v, [lo, dvec])
            dl = plsc.load_gather(dl_v, [lo, dvec])
            da = plsc.load_gather(dv_v, [lo, dvec])
            db = plsc.load_gather(dv_v, [lo1, dvec])

            w = cw1 - cw
            theta = (xc - cw) / w
            t2 = theta * theta
            t1m = theta - t2
            num = (dl * w) * (dl * t2 + da * t1m)
            den = dl + (da + db - 2.0 * dl) * t1m
            inv = 1.0 / den
            out_s = chh + num * inv
            omt = 1.0 - theta
            dnum = (dl * dl) * (db * t2 + 2.0 * dl * t1m + da * omt * omt)
            ob[r, pl.ds(j * L, L)] = jnp.where(inside, out_s, x0)

            ratio = jnp.where(inside, dnum * inv * inv, 1.0)
            pacc = pacc * ratio
            bits = plsc.bitcast(pacc, jnp.int32)
            eacc = eacc + (lax.shift_right_arithmetic(bits, 23) - 127)
            mant = (bits & 0x007FFFFF) | 0x3F800000
            pacc = plsc.bitcast(mant, jnp.float32)
            return pacc, eacc

        def make_row_body(xb, ob):
            def row_body(r, lad_acc):
                init = (jnp.ones((L,), jnp.float32),
                        jnp.zeros((L,), jnp.int32))

                @plsc.parallel_loop(0, NVEC, carry=init, unroll=4)
                def final_carry(j, c):
                    return vec_body(xb, ob, r, j, c)

                pacc, eacc = final_carry
                lad = _ln_1_2(pacc) + eacc.astype(jnp.float32) * LN2
                s = jnp.sum(lad)
                return lad_acc + jnp.where(lane == r, s, 0.0)
            return row_body

        out_handles = [None, None]
        for c in range(NCHUNK):
            cur = c % 2
            base = row0 + c * R
            if c == 0:
                in_h = first_in
            if c + 1 < NCHUNK:
                next_in = pltpu.async_copy(
                    x_hbm.at[pl.ds(base + R, R)], xbuf.at[1 - cur],
                    in_sems[1 - cur])
            in_h.wait()
            if out_handles[cur] is not None:
                for h in out_handles[cur]:
                    h.wait()
            lad_acc = lax.fori_loop(
                0, R, make_row_body(xbuf.at[cur], obuf.at[cur]),
                jnp.zeros((L,), jnp.float32))
            ladbuf[cur, ...] = lad_acc
            out_handles[cur] = [
                pltpu.async_copy(obuf.at[cur], out_hbm.at[pl.ds(base, R)],
                                 out_sems[cur]),
                pltpu.async_copy(ladbuf.at[cur], lad_hbm.at[pl.ds(base, R)],
                                 out_sems[cur]),
            ]
            if c + 1 < NCHUNK:
                in_h = next_in
        for hs in out_handles:
            if hs is not None:
                for h in hs:
                    h.wait()

    return sc_kernel(x, edges_t, ch_t, dl_t, dv_t)


@jax.jit
def kernel(inputs, unnormalized_widths, unnormalized_heights,
           unnormalized_derivatives):
    tables = _prep_tables(unnormalized_widths, unnormalized_heights,
                          unnormalized_derivatives)
    out, lad = _sc_main(inputs, tables)
    return out, lad
